# baseline (speedup 1.0000x reference)
import functools

import jax
import jax.numpy as jnp
from jax import lax
from jax.experimental import pallas as pl
from jax.experimental.pallas import tpu as pltpu
from jax.experimental.pallas import tpu_sc as plsc

_MASK_RATIO = 0.15
_B, _L, _D = 16, 2048, 1024
_LEN_KEEP = int(_L * (1 - _MASK_RATIO))
_R = _B * _LEN_KEEP
_NDROP = _B * (_L - _LEN_KEEP)

_NC, _NS = 2, 16
_NW = _NC * _NS
_W = 40
_KCH = _R // _W
_DCH = -(-_NDROP // _W)
_TCH = _KCH + _DCH
_SLOTS = -(-_TCH // _NW)
_IDXR = 32

_MFULL = _KCH // _NW
_MREM = _KCH - _MFULL * _NW
_SFULL = _TCH // _NW
_SREM = _TCH - _SFULL * _NW


def _index_arrays():
    noise = jax.random.uniform(
        jax.random.fold_in(jax.random.key(0), 1), (_B, _L),
        dtype=jnp.float32)
    shuf = jnp.argsort(noise, axis=1).astype(jnp.int32)
    base = (jnp.arange(_B, dtype=jnp.int32) * _L)[:, None]
    kept = (shuf[:, :_LEN_KEEP] + base).T.reshape(-1)
    drop = (shuf[:, _LEN_KEEP:] + base).reshape(-1)
    drop = jnp.pad(drop, (0, _DCH * _W - _NDROP), mode="edge")
    chunks = jnp.concatenate([kept, drop]).reshape(_TCH, _W)
    chunks = jnp.pad(chunks, ((0, _SLOTS * _NW - _TCH), (0, 0)))
    sidx = jnp.arange(_NW)[:, None] + jnp.arange(_SLOTS)[None, :] * _NW
    per_w = chunks[sidx]
    per_w = jnp.pad(per_w, ((0, 0), (0, _IDXR - _SLOTS), (0, 0)))
    return per_w.reshape(_NW * _IDXR, _W), shuf.reshape(-1)


def _body(x_hbm, gidx_hbm, shuf_hbm, out_hbm, xout_hbm, restore_hbm,
          idx_v, rows_v, shuf_v, rest_v,
          gsem0, gsem1, gsem2, wsem0, wsem1, wsem2, ssem0, ssem1, ssem2):
    cid = lax.axis_index("c")
    sid = lax.axis_index("s")
    wid = sid * _NC + cid
    gsem = (gsem0, gsem1, gsem2)
    wsem = (wsem0, wsem1, wsem2)
    ssem = (ssem0, ssem1, ssem2)

    pltpu.sync_copy(gidx_hbm.at[pl.ds(wid * _IDXR, _IDXR)], idx_v)

    def gcopy(s):
        b = s % 3
        return pltpu.make_async_copy(
            x_hbm.at[idx_v.at[s]], rows_v.at[b], gsem[b])

    def mcopy(s):
        b = s % 3
        g = s * _NW + wid
        return pltpu.make_async_copy(rows_v.at[b], out_hbm.at[g], wsem[b])

    def scopy(s):
        b = s % 3
        return pltpu.make_async_copy(
            rows_v.at[b], xout_hbm.at[idx_v.at[s]], ssem[b])

    def mguard(s, fn):
        if s < _MFULL:
            fn()
        elif s == _MFULL:
            pl.when(wid < _MREM)(fn)

    def sguard(s, fn):
        if s < _SFULL:
            fn()
        elif s == _SFULL:
            pl.when(wid < _SREM)(fn)

    for s in range(3):
        gcopy(s).start()
    for s in range(_SLOTS):
        gcopy(s).wait()
        mguard(s, lambda s=s: mcopy(s).start())
        sguard(s, lambda s=s: scopy(s).start())
        n = s + 3
        if n < _SLOTS:
            mguard(s, lambda s=s: mcopy(s).wait())
            sguard(s, lambda s=s: scopy(s).wait())
            gcopy(n).start()
    for s in range(_SLOTS - 3, _SLOTS):
        mguard(s, lambda s=s: mcopy(s).wait())
        sguard(s, lambda s=s: scopy(s).wait())

    @pl.when(wid < _B)
    def _():
        pltpu.sync_copy(shuf_hbm.at[pl.ds(wid * _L, _L)], shuf_v)

        def it(i, carry):
            tgt = shuf_v[pl.ds(i * 16, 16)]
            vals = lax.iota(jnp.int32, 16) + i * 16
            plsc.store_scatter(rest_v, [tgt], vals)
            return carry

        lax.fori_loop(0, _L // 16, it, 0)
        pltpu.sync_copy(rest_v, restore_hbm.at[pl.ds(wid * _L, _L)])


@functools.cache
def _sc_gather():
    return pl.kernel(
        _body,
        out_type=(
            jax.ShapeDtypeStruct((_KCH, _W, _D), jnp.float32),
            jax.ShapeDtypeStruct((_B * _L, _D), jnp.float32),
            jax.ShapeDtypeStruct((_B * _L,), jnp.int32),
        ),
        mesh=plsc.VectorSubcoreMesh(core_axis_name="c", subcore_axis_name="s"),
        compiler_params=pltpu.CompilerParams(
            needs_layout_passes=False, use_tc_tiling_on_sc=True),
        scratch_types=(
            pltpu.VMEM((_IDXR, _W), jnp.int32),
            pltpu.VMEM((3, _W, _D), jnp.float32),
            pltpu.VMEM((_L,), jnp.int32),
            pltpu.VMEM((_L,), jnp.int32),
            pltpu.SemaphoreType.DMA,
            pltpu.SemaphoreType.DMA,
            pltpu.SemaphoreType.DMA,
            pltpu.SemaphoreType.DMA,
            pltpu.SemaphoreType.DMA,
            pltpu.SemaphoreType.DMA,
            pltpu.SemaphoreType.DMA,
            pltpu.SemaphoreType.DMA,
            pltpu.SemaphoreType.DMA,
        ),
    )


def kernel(x):
    gidx, shuf = _index_arrays()
    x_flat = x.reshape(_B * _L, _D)
    out_flat, x_out, restore = _sc_gather()(x_flat, gidx, shuf)
    x_masked = out_flat.reshape(_LEN_KEEP, _B, _D).transpose(1, 0, 2)
    return x_masked, x_out.reshape(_B, _L, _D), restore.reshape(_B, _L)


# speedup vs baseline: 1.0418x; 1.0418x over previous
import functools

import jax
import jax.numpy as jnp
from jax import lax
from jax.experimental import pallas as pl
from jax.experimental.pallas import tpu as pltpu
from jax.experimental.pallas import tpu_sc as plsc

_MASK_RATIO = 0.15
_B, _L, _D = 16, 2048, 1024
_LEN_KEEP = int(_L * (1 - _MASK_RATIO))
_R = _B * _LEN_KEEP

_NC, _NS = 2, 16
_NW = _NC * _NS
_W = 40
_GCH = _R // _W
_SLOTS = -(-_GCH // _NW)
_PW = _SLOTS * _W


def _index_arrays():
    noise = jax.random.uniform(
        jax.random.fold_in(jax.random.key(0), 1), (_B, _L),
        dtype=jnp.float32)
    shuf = jnp.argsort(noise, axis=1).astype(jnp.int32)
    keep = shuf[:, :_LEN_KEEP]
    gidx = keep + (jnp.arange(_B, dtype=jnp.int32) * _L)[:, None]
    flat = gidx.T.reshape(-1)
    flat = jnp.pad(flat, (0, _SLOTS * _NW * _W - _R))
    chunks = flat.reshape(_SLOTS * _NW, _W)
    order = (jnp.arange(_NW)[:, None] + jnp.arange(_SLOTS)[None, :] * _NW)
    per_w = chunks[order.reshape(-1)]
    return per_w.reshape(-1), shuf.reshape(-1)


def _body(x_hbm, gidx_hbm, shuf_hbm, out_hbm, restore_hbm,
          idx_v, rows_v, shuf_v, rest_v,
          gsem0, gsem1, gsem2, wsem0, wsem1, wsem2):
    cid = lax.axis_index("c")
    sid = lax.axis_index("s")
    wid = sid * _NC + cid
    gsem = (gsem0, gsem1, gsem2)
    wsem = (wsem0, wsem1, wsem2)

    pltpu.sync_copy(gidx_hbm.at[pl.ds(wid * _PW, _PW)], idx_v)

    def gcopy(s):
        b = s % 3
        return pltpu.make_async_copy(
            x_hbm.at[idx_v.at[pl.ds(s * _W, _W)]], rows_v.at[b], gsem[b])

    def wcopy(s):
        b = s % 3
        g = s * _NW + wid
        return pltpu.make_async_copy(rows_v.at[b], out_hbm.at[g], wsem[b])

    _REM = _GCH - (_SLOTS - 1) * _NW

    for s in range(3):
        gcopy(s).start()
    for s in range(_SLOTS):

        def step(s=s):
            gcopy(s).wait()
            wcopy(s).start()
            n = s + 3
            if n < _SLOTS - 1:
                wcopy(n - 3).wait()
                gcopy(n).start()
            elif n == _SLOTS - 1:
                @pl.when(wid < _REM)
                def _(n=n):
                    wcopy(n - 3).wait()
                    gcopy(n).start()

        if s == _SLOTS - 1:
            pl.when(wid < _REM)(step)
        else:
            step()

    for s in (_SLOTS - 3, _SLOTS - 2):
        wcopy(s).wait()
    pl.when(wid < _REM)(lambda: wcopy(_SLOTS - 1).wait())
    pl.when(wid >= _REM)(lambda: wcopy(_SLOTS - 4).wait())

    @pl.when(wid < _B)
    def _():
        pltpu.sync_copy(shuf_hbm.at[pl.ds(wid * _L, _L)], shuf_v)

        def it(i, carry):
            tgt = shuf_v[pl.ds(i * 16, 16)]
            vals = lax.iota(jnp.int32, 16) + i * 16
            plsc.store_scatter(rest_v, [tgt], vals)
            return carry

        lax.fori_loop(0, _L // 16, it, 0)
        pltpu.sync_copy(rest_v, restore_hbm.at[pl.ds(wid * _L, _L)])


@functools.cache
def _sc_gather():
    return pl.kernel(
        _body,
        out_type=(
            jax.ShapeDtypeStruct((_GCH, _W, _D), jnp.float32),
            jax.ShapeDtypeStruct((_B * _L,), jnp.int32),
        ),
        mesh=plsc.VectorSubcoreMesh(core_axis_name="c", subcore_axis_name="s"),
        compiler_params=pltpu.CompilerParams(
            needs_layout_passes=False, use_tc_tiling_on_sc=True),
        scratch_types=(
            pltpu.VMEM((_PW,), jnp.int32),
            pltpu.VMEM((3, _W, _D), jnp.float32),
            pltpu.VMEM((_L,), jnp.int32),
            pltpu.VMEM((_L,), jnp.int32),
            pltpu.SemaphoreType.DMA,
            pltpu.SemaphoreType.DMA,
            pltpu.SemaphoreType.DMA,
            pltpu.SemaphoreType.DMA,
            pltpu.SemaphoreType.DMA,
            pltpu.SemaphoreType.DMA,
        ),
    )


def _tc_copy_body(x_ref, o_ref):
    o_ref[...] = x_ref[...]


_XBLK = 512


@functools.cache
def _tc_copy():
    return pl.pallas_call(
        _tc_copy_body,
        out_shape=jax.ShapeDtypeStruct((_B * _L, _D), jnp.float32),
        grid=(_B * _L // _XBLK,),
        in_specs=[pl.BlockSpec((_XBLK, _D), lambda i: (i, 0))],
        out_specs=pl.BlockSpec((_XBLK, _D), lambda i: (i, 0)),
    )


def kernel(x):
    gidx, shuf = _index_arrays()
    x_flat = x.reshape(_B * _L, _D)
    out_flat, restore = _sc_gather()(x_flat, gidx, shuf)
    x_out = _tc_copy()(x_flat)
    x_masked = out_flat.reshape(_LEN_KEEP, _B, _D).transpose(1, 0, 2)
    return x_masked, x_out.reshape(_B, _L, _D), restore.reshape(_B, _L)
